# baseline (speedup 1.0000x reference)
import functools

import jax
import jax.numpy as jnp
from jax import lax
from jax.experimental import pallas as pl
from jax.experimental.pallas import tpu as pltpu
from jax.experimental.pallas import tpu_sc as plsc

N_TOKENS = 1048576
VOCAB = 1000000
EMBED_DIM = 32
VOCABX = VOCAB * EMBED_DIM
NUM_CLASS = 4
BATCH = 16
SEG = N_TOKENS // BATCH
HALF = SEG // 2
CHUNK = 1024
SUBCHUNK = 128
N_SUB = CHUNK // SUBCHUNK
N_STEPS = HALF // CHUNK


def _seg_sum_kernel(text2d, emb_table):
    mesh = plsc.VectorSubcoreMesh(core_axis_name="c", subcore_axis_name="s")

    @functools.partial(
        pl.kernel,
        mesh=mesh,
        out_type=jax.ShapeDtypeStruct((32, EMBED_DIM), jnp.float32),
        scratch_types=[
            pltpu.VMEM((2, N_SUB, SUBCHUNK), jnp.int32),
            pltpu.VMEM((2, CHUNK, EMBED_DIM), jnp.float32),
            pltpu.VMEM((EMBED_DIM,), jnp.float32),
            pltpu.SemaphoreType.DMA,
            pltpu.SemaphoreType.DMA,
        ],
        compiler_params=pltpu.CompilerParams(use_tc_tiling_on_sc=False),
    )
    def body(text_hbm, emb_hbm, out_hbm, idx_v, rows_v, acc_v, isem, gsem):
        c = lax.axis_index("c")
        s = lax.axis_index("s")
        wid = c * 16 + s
        row_base = s * (SEG // SUBCHUNK) + c * (HALF // SUBCHUNK)
        last = N_STEPS - 1

        def idx_copy(step, b):
            rb = row_base + lax.min(step, last) * N_SUB
            return pltpu.make_async_copy(
                text_hbm.at[pl.ds(rb, N_SUB)], idx_v.at[b], isem
            )

        def gather(b, j):
            return pltpu.make_async_copy(
                emb_hbm.at[idx_v.at[b].at[j]],
                rows_v.at[b].at[pl.ds(j * SUBCHUNK, SUBCHUNK)],
                gsem,
            )

        idx_copy(0, 0).start()
        idx_copy(0, 0).wait()
        for j in range(N_SUB):
            gather(0, j).start()
        idx_copy(1, 1).start()

        zero = jnp.zeros((16,), jnp.float32)

        def accum(b, carry):
            def rows8(r, c2):
                b0, b1 = c2
                base = r * 8
                for u in range(8):
                    b0 = b0 + rows_v[b, base + u, pl.ds(0, 16)]
                    b1 = b1 + rows_v[b, base + u, pl.ds(16, 16)]
                return b0, b1

            return lax.fori_loop(0, CHUNK // 8, rows8, carry)

        def pair(g, carry):
            for b in (0, 1):
                i = g * 2 + b
                idx_copy(i + 1, 1 - b).wait()
                for j in range(N_SUB):
                    gather(1 - b, j).start()
                for j in range(N_SUB):
                    gather(b, j).wait()
                idx_copy(i + 2, b).start()
                carry = accum(b, carry)
            return carry

        a0, a1 = lax.fori_loop(0, N_STEPS // 2, pair, (zero, zero))

        idx_copy(N_STEPS + 1, 1).wait()
        for j in range(N_SUB):
            gather(0, j).wait()

        acc_v[pl.ds(0, 16)] = a0
        acc_v[pl.ds(16, 16)] = a1
        pltpu.sync_copy(acc_v, out_hbm.at[wid])

    return body(text2d, emb_table)


TR_BLK = 2048
TR_I = 123
SLAB = TR_BLK * TR_I


def _transpose_body(t0, t1, t2, t3, o_ref):
    tcat = jnp.concatenate([t0[...], t1[...], t2[...], t3[...]], axis=0)
    n = 4 * EMBED_DIM
    rows = lax.broadcasted_iota(jnp.int32, (n, n), 0)
    cols = lax.broadcasted_iota(jnp.int32, (n, n), 1)
    ident = (cols == rows).astype(jnp.float32)
    o_ref[...] = lax.dot_general(
        tcat, ident, (((0,), (0,)), ((), ())),
        precision=lax.Precision.HIGHEST,
        preferred_element_type=jnp.float32,
    )


def _linearize_table(emb_table):
    emb_t = emb_table.T
    lin = pl.pallas_call(
        _transpose_body,
        grid=(TR_I,),
        in_specs=[
            pl.BlockSpec(
                (EMBED_DIM, TR_BLK),
                lambda i, j=j: (0, jnp.minimum(j * TR_I + i, VOCAB // TR_BLK)),
            )
            for j in range(4)
        ],
        out_specs=pl.BlockSpec((TR_BLK, 4 * EMBED_DIM), lambda i: (i, 0)),
        out_shape=jax.ShapeDtypeStruct((SLAB, 4 * EMBED_DIM), jnp.float32),
    )(emb_t, emb_t, emb_t, emb_t)
    return lin.reshape(4 * SLAB, EMBED_DIM)


def _head_body(p_ref, w_ref, b_ref, o_ref):
    p = p_ref[...]
    pooled = (p[0:16, :] + p[16:32, :]) * (1.0 / SEG)
    o_ref[...] = (
        lax.dot_general(
            pooled, w_ref[...], (((1,), (1,)), ((), ())),
            preferred_element_type=jnp.float32,
        )
        + b_ref[...]
    )


def kernel(text, emb_table, fc_w, fc_b):
    emb_lin = _linearize_table(emb_table)
    slab_id = (
        (text >= SLAB).astype(jnp.int32)
        + (text >= 2 * SLAB).astype(jnp.int32)
        + (text >= 3 * SLAB).astype(jnp.int32)
    )
    text_r = 4 * text - slab_id * (4 * SLAB - 1)
    text2d = text_r.reshape(N_TOKENS // SUBCHUNK, SUBCHUNK)
    partials = _seg_sum_kernel(text2d, emb_lin)
    return pl.pallas_call(
        _head_body,
        out_shape=jax.ShapeDtypeStruct((BATCH, NUM_CLASS), jnp.float32),
    )(partials, fc_w, fc_b.reshape(1, NUM_CLASS))


# speedup vs baseline: 1.1532x; 1.1532x over previous
import functools

import jax
import jax.numpy as jnp
from jax import lax
from jax.experimental import pallas as pl
from jax.experimental.pallas import tpu as pltpu
from jax.experimental.pallas import tpu_sc as plsc

N_TOKENS = 1048576
VOCAB = 1000000
EMBED_DIM = 32
VOCABX = VOCAB * EMBED_DIM
NUM_CLASS = 4
BATCH = 16
SEG = N_TOKENS // BATCH
HALF = SEG // 2
CHUNK = 1024
SUBCHUNK = 128
N_SUB = CHUNK // SUBCHUNK
N_STEPS = HALF // CHUNK


def _seg_sum_kernel(text2d, emb_table):
    mesh = plsc.VectorSubcoreMesh(core_axis_name="c", subcore_axis_name="s")

    @functools.partial(
        pl.kernel,
        mesh=mesh,
        out_type=jax.ShapeDtypeStruct((32, EMBED_DIM), jnp.float32),
        scratch_types=[
            pltpu.VMEM((2, N_SUB, SUBCHUNK), jnp.int32),
            pltpu.VMEM((2, CHUNK, EMBED_DIM), jnp.float32),
            pltpu.VMEM((EMBED_DIM,), jnp.float32),
            pltpu.SemaphoreType.DMA,
            pltpu.SemaphoreType.DMA,
        ],
        compiler_params=pltpu.CompilerParams(use_tc_tiling_on_sc=False),
    )
    def body(text_hbm, emb_hbm, out_hbm, idx_v, rows_v, acc_v, isem, gsem):
        c = lax.axis_index("c")
        s = lax.axis_index("s")
        wid = c * 16 + s
        row_base = s * (SEG // SUBCHUNK) + c * (HALF // SUBCHUNK)
        last = N_STEPS - 1

        def idx_copy(step, b):
            rb = row_base + lax.min(step, last) * N_SUB
            return pltpu.make_async_copy(
                text_hbm.at[pl.ds(rb, N_SUB)], idx_v.at[b], isem
            )

        def gather(b, j):
            return pltpu.make_async_copy(
                emb_hbm.at[idx_v.at[b].at[j]],
                rows_v.at[b].at[pl.ds(j * SUBCHUNK, SUBCHUNK)],
                gsem,
            )

        idx_copy(0, 0).start()
        idx_copy(0, 0).wait()
        for j in range(N_SUB):
            gather(0, j).start()
        idx_copy(1, 1).start()

        zero = jnp.zeros((16,), jnp.float32)

        def accum(b, carry):
            def rows8(r, c2):
                b0, b1 = c2
                base = r * 8
                for u in range(8):
                    b0 = b0 + rows_v[b, base + u, pl.ds(0, 16)]
                    b1 = b1 + rows_v[b, base + u, pl.ds(16, 16)]
                return b0, b1

            return lax.fori_loop(0, CHUNK // 8, rows8, carry)

        def pair(g, carry):
            for b in (0, 1):
                i = g * 2 + b
                idx_copy(i + 1, 1 - b).wait()
                for j in range(N_SUB):
                    gather(1 - b, j).start()
                for j in range(N_SUB):
                    gather(b, j).wait()
                idx_copy(i + 2, b).start()
                carry = accum(b, carry)
            return carry

        a0, a1 = lax.fori_loop(0, N_STEPS // 2, pair, (zero, zero))

        idx_copy(N_STEPS + 1, 1).wait()
        for j in range(N_SUB):
            gather(0, j).wait()

        acc_v[pl.ds(0, 16)] = a0
        acc_v[pl.ds(16, 16)] = a1
        pltpu.sync_copy(acc_v, out_hbm.at[wid])

    return body(text2d, emb_table)


TR_BLK = 2048
TR_I = 123
SLAB = TR_BLK * TR_I


def _transpose_body(t0, t1, t2, t3, o_ref):
    tcat = jnp.concatenate([t0[...], t1[...], t2[...], t3[...]], axis=0)
    n = 4 * EMBED_DIM
    rows = lax.broadcasted_iota(jnp.int32, (n, n), 0)
    cols = lax.broadcasted_iota(jnp.int32, (n, n), 1)
    ident = (cols == rows).astype(jnp.float32)
    o_ref[...] = lax.dot_general(
        tcat, ident, (((0,), (0,)), ((), ())),
        preferred_element_type=jnp.float32,
    )


def _linearize_table(emb_table):
    emb_t = emb_table.T
    lin = pl.pallas_call(
        _transpose_body,
        grid=(TR_I,),
        in_specs=[
            pl.BlockSpec(
                (EMBED_DIM, TR_BLK),
                lambda i, j=j: (0, jnp.minimum(j * TR_I + i, VOCAB // TR_BLK)),
            )
            for j in range(4)
        ],
        out_specs=pl.BlockSpec((TR_BLK, 4 * EMBED_DIM), lambda i: (i, 0)),
        out_shape=jax.ShapeDtypeStruct((SLAB, 4 * EMBED_DIM), jnp.float32),
    )(emb_t, emb_t, emb_t, emb_t)
    return lin.reshape(4 * SLAB, EMBED_DIM)


def _head_body(p_ref, w_ref, b_ref, o_ref):
    p = p_ref[...]
    pooled = (p[0:16, :] + p[16:32, :]) * (1.0 / SEG)
    o_ref[...] = (
        lax.dot_general(
            pooled, w_ref[...], (((1,), (1,)), ((), ())),
            preferred_element_type=jnp.float32,
        )
        + b_ref[...]
    )


def kernel(text, emb_table, fc_w, fc_b):
    emb_lin = _linearize_table(emb_table)
    slab_id = (
        (text >= SLAB).astype(jnp.int32)
        + (text >= 2 * SLAB).astype(jnp.int32)
        + (text >= 3 * SLAB).astype(jnp.int32)
    )
    text_r = 4 * text - slab_id * (4 * SLAB - 1)
    text2d = text_r.reshape(N_TOKENS // SUBCHUNK, SUBCHUNK)
    partials = _seg_sum_kernel(text2d, emb_lin)
    return pl.pallas_call(
        _head_body,
        out_shape=jax.ShapeDtypeStruct((BATCH, NUM_CLASS), jnp.float32),
    )(partials, fc_w, fc_b.reshape(1, NUM_CLASS))


# speedup vs baseline: 1.1732x; 1.0173x over previous
import functools

import jax
import jax.numpy as jnp
from jax import lax
from jax.experimental import pallas as pl
from jax.experimental.pallas import tpu as pltpu
from jax.experimental.pallas import tpu_sc as plsc

N_TOKENS = 1048576
VOCAB = 1000000
EMBED_DIM = 32
VOCABX = VOCAB * EMBED_DIM
NUM_CLASS = 4
BATCH = 16
SEG = N_TOKENS // BATCH
HALF = SEG // 2
CHUNK = 1024
SUBCHUNK = 128
N_SUB = CHUNK // SUBCHUNK
N_STEPS = HALF // CHUNK


def _seg_sum_kernel(text2d, emb_table):
    mesh = plsc.VectorSubcoreMesh(core_axis_name="c", subcore_axis_name="s")

    @functools.partial(
        pl.kernel,
        mesh=mesh,
        out_type=jax.ShapeDtypeStruct((32, EMBED_DIM), jnp.float32),
        scratch_types=[
            pltpu.VMEM((2, N_SUB, SUBCHUNK), jnp.int32),
            pltpu.VMEM((2, CHUNK, EMBED_DIM), jnp.float32),
            pltpu.VMEM((EMBED_DIM,), jnp.float32),
            pltpu.SemaphoreType.DMA,
            pltpu.SemaphoreType.DMA,
        ],
        compiler_params=pltpu.CompilerParams(use_tc_tiling_on_sc=False),
    )
    def body(text_hbm, emb_hbm, out_hbm, idx_v, rows_v, acc_v, isem, gsem):
        c = lax.axis_index("c")
        s = lax.axis_index("s")
        wid = c * 16 + s
        row_base = s * (SEG // SUBCHUNK) + c * (HALF // SUBCHUNK)
        last = N_STEPS - 1

        def idx_copy(step, b):
            rb = row_base + lax.min(step, last) * N_SUB
            return pltpu.make_async_copy(
                text_hbm.at[pl.ds(rb, N_SUB)], idx_v.at[b], isem
            )

        def gather(b, j):
            return pltpu.make_async_copy(
                emb_hbm.at[idx_v.at[b].at[j]],
                rows_v.at[b].at[pl.ds(j * SUBCHUNK, SUBCHUNK)],
                gsem,
            )

        idx_copy(0, 0).start()
        idx_copy(0, 0).wait()
        for j in range(N_SUB):
            gather(0, j).start()
        idx_copy(1, 1).start()

        zero = jnp.zeros((16,), jnp.float32)

        def accum(b, carry):
            def rows8(r, c2):
                b0, b1 = c2
                base = r * 8
                for u in range(8):
                    b0 = b0 + rows_v[b, base + u, pl.ds(0, 16)]
                    b1 = b1 + rows_v[b, base + u, pl.ds(16, 16)]
                return b0, b1

            return lax.fori_loop(0, CHUNK // 8, rows8, carry)

        def pair(g, carry):
            for b in (0, 1):
                i = g * 2 + b
                idx_copy(i + 1, 1 - b).wait()
                for j in range(N_SUB):
                    gather(1 - b, j).start()
                for j in range(N_SUB):
                    gather(b, j).wait()
                idx_copy(i + 2, b).start()
                carry = accum(b, carry)
            return carry

        a0, a1 = lax.fori_loop(0, N_STEPS // 2, pair, (zero, zero))

        idx_copy(N_STEPS + 1, 1).wait()
        for j in range(N_SUB):
            gather(0, j).wait()

        acc_v[pl.ds(0, 16)] = a0
        acc_v[pl.ds(16, 16)] = a1
        pltpu.sync_copy(acc_v, out_hbm.at[wid])

    return body(text2d, emb_table)


TR_SUB = 2048
TR_BLK = 4 * TR_SUB
TR_I = 123


def _transpose_body(t_ref, o_ref):
    t = t_ref[...]
    tcat = jnp.concatenate(
        [t[:, j * TR_SUB:(j + 1) * TR_SUB] for j in range(4)], axis=0
    )
    n = 4 * EMBED_DIM
    rows = lax.broadcasted_iota(jnp.int32, (n, n), 0)
    cols = lax.broadcasted_iota(jnp.int32, (n, n), 1)
    ident = (cols == rows).astype(jnp.float32)
    o_ref[...] = lax.dot_general(
        tcat, ident, (((0,), (0,)), ((), ())),
        preferred_element_type=jnp.float32,
    )


def _linearize_table(emb_table):
    emb_t = emb_table.T
    lin = pl.pallas_call(
        _transpose_body,
        grid=(TR_I,),
        in_specs=[pl.BlockSpec((EMBED_DIM, TR_BLK), lambda i: (0, i))],
        out_specs=pl.BlockSpec((TR_SUB, 4 * EMBED_DIM), lambda i: (i, 0)),
        out_shape=jax.ShapeDtypeStruct((TR_I * TR_SUB, 4 * EMBED_DIM),
                                       jnp.float32),
    )(emb_t)
    return lin.reshape(TR_I * TR_BLK, EMBED_DIM)


def _head_body(p_ref, w_ref, b_ref, o_ref):
    p = p_ref[...]
    pooled = (p[0:16, :] + p[16:32, :]) * (1.0 / SEG)
    o_ref[...] = (
        lax.dot_general(
            pooled, w_ref[...], (((1,), (1,)), ((), ())),
            preferred_element_type=jnp.float32,
        )
        + b_ref[...]
    )


def kernel(text, emb_table, fc_w, fc_b):
    emb_lin = _linearize_table(emb_table)
    text_r = (
        (text & ~(TR_BLK - 1))
        + ((text & (TR_SUB - 1)) << 2)
        + ((text >> 11) & 3)
    )
    text2d = text_r.reshape(N_TOKENS // SUBCHUNK, SUBCHUNK)
    partials = _seg_sum_kernel(text2d, emb_lin)
    return pl.pallas_call(
        _head_body,
        out_shape=jax.ShapeDtypeStruct((BATCH, NUM_CLASS), jnp.float32),
    )(partials, fc_w, fc_b.reshape(1, NUM_CLASS))


# speedup vs baseline: 1.1879x; 1.0125x over previous
import functools

import jax
import jax.numpy as jnp
from jax import lax
from jax.experimental import pallas as pl
from jax.experimental.pallas import tpu as pltpu
from jax.experimental.pallas import tpu_sc as plsc

N_TOKENS = 1048576
VOCAB = 1000000
EMBED_DIM = 32
NUM_CLASS = 4
BATCH = 16
SEG = N_TOKENS // BATCH
HALF = SEG // 2
CHUNK = 1024
SUBCHUNK = 128
N_SUB = CHUNK // SUBCHUNK
N_STEPS = HALF // CHUNK
ROW_I32 = EMBED_DIM // 2

TR_BLK = 8192
TR_SUB = TR_BLK // 8
TR_I = 123


def _relayout_body(t_ref, o_ref):
    t = t_ref[...]
    tcat = jnp.concatenate(
        [t[:, m * TR_SUB:(m + 1) * TR_SUB] for m in range(8)], axis=0
    )
    rows = lax.broadcasted_iota(jnp.int32, (8 * EMBED_DIM, 4 * EMBED_DIM), 0)
    cols = lax.broadcasted_iota(jnp.int32, (8 * EMBED_DIM, 4 * EMBED_DIM), 1)
    s_even = (rows == 2 * cols).astype(jnp.float32)
    s_odd = (rows == 2 * cols + 1).astype(jnp.float32)
    dims = (((0,), (0,)), ((), ()))
    lo = lax.dot_general(tcat, s_even, dims, preferred_element_type=jnp.float32)
    hi = lax.dot_general(tcat, s_odd, dims, preferred_element_type=jnp.float32)

    def b16(f):
        u = lax.bitcast_convert_type(f, jnp.int32)
        lsb = lax.shift_right_logical(u, 16) & 1
        return lax.shift_right_logical(u + (0x7FFF + lsb), 16)

    o_ref[...] = b16(lo) | lax.shift_left(b16(hi), 16)


def _linearize_table(emb_table):
    emb_t = emb_table.T
    lin = pl.pallas_call(
        _relayout_body,
        grid=(TR_I,),
        in_specs=[pl.BlockSpec((EMBED_DIM, TR_BLK), lambda i: (0, i))],
        out_specs=pl.BlockSpec((TR_SUB, 4 * EMBED_DIM), lambda i: (i, 0)),
        out_shape=jax.ShapeDtypeStruct((TR_I * TR_SUB, 4 * EMBED_DIM),
                                       jnp.int32),
    )(emb_t)
    return lin.reshape(TR_I * TR_BLK, ROW_I32)


def _seg_sum_kernel(text2d, emb_lin):
    mesh = plsc.VectorSubcoreMesh(core_axis_name="c", subcore_axis_name="s")

    @functools.partial(
        pl.kernel,
        mesh=mesh,
        out_type=jax.ShapeDtypeStruct((32, EMBED_DIM), jnp.float32),
        scratch_types=[
            pltpu.VMEM((2, N_SUB, SUBCHUNK), jnp.int32),
            pltpu.VMEM((2, CHUNK, ROW_I32), jnp.int32),
            pltpu.VMEM((EMBED_DIM,), jnp.float32),
            pltpu.SemaphoreType.DMA,
            pltpu.SemaphoreType.DMA,
        ],
        compiler_params=pltpu.CompilerParams(use_tc_tiling_on_sc=False),
    )
    def body(text_hbm, emb_hbm, out_hbm, idx_v, rows_v, acc_v, isem, gsem):
        c = lax.axis_index("c")
        s = lax.axis_index("s")
        wid = c * 16 + s
        row_base = s * (SEG // SUBCHUNK) + c * (HALF // SUBCHUNK)
        last = N_STEPS - 1

        def idx_copy(step, b):
            rb = row_base + lax.min(step, last) * N_SUB
            return pltpu.make_async_copy(
                text_hbm.at[pl.ds(rb, N_SUB)], idx_v.at[b], isem
            )

        def gather(b, j):
            return pltpu.make_async_copy(
                emb_hbm.at[idx_v.at[b].at[j]],
                rows_v.at[b].at[pl.ds(j * SUBCHUNK, SUBCHUNK)],
                gsem,
            )

        idx_copy(0, 0).start()
        idx_copy(0, 0).wait()
        for j in range(N_SUB):
            gather(0, j).start()
        idx_copy(1, 1).start()

        zero = jnp.zeros((16,), jnp.float32)

        def accum(b, carry):
            def rows8(r, c2):
                b0, b1 = c2
                base = r * 8
                for u in range(8):
                    w = rows_v[b, base + u, :]
                    ea = lax.bitcast_convert_type(
                        lax.shift_left(w, 16), jnp.float32
                    )
                    eb = lax.bitcast_convert_type(
                        w & jnp.int32(-65536), jnp.float32
                    )
                    b0 = b0 + ea
                    b1 = b1 + eb
                return b0, b1

            return lax.fori_loop(0, CHUNK // 8, rows8, carry)

        def pair(g, carry):
            for b in (0, 1):
                i = g * 2 + b
                idx_copy(i + 1, 1 - b).wait()
                for j in range(N_SUB):
                    gather(1 - b, j).start()
                for j in range(N_SUB):
                    gather(b, j).wait()
                idx_copy(i + 2, b).start()
                carry = accum(b, carry)
            return carry

        a0, a1 = lax.fori_loop(0, N_STEPS // 2, pair, (zero, zero))

        idx_copy(N_STEPS + 1, 1).wait()
        for j in range(N_SUB):
            gather(0, j).wait()

        acc_v[pl.ds(0, 16)] = a0
        acc_v[pl.ds(16, 16)] = a1
        pltpu.sync_copy(acc_v, out_hbm.at[wid])

    return body(text2d, emb_lin)


def _head_body(p_ref, w_ref, b_ref, o_ref):
    p = p_ref[...]
    pooled = (p[0:16, :] + p[16:32, :]) * (1.0 / SEG)
    o_ref[...] = (
        lax.dot_general(
            pooled, w_ref[...], (((1,), (1,)), ((), ())),
            preferred_element_type=jnp.float32,
        )
        + b_ref[...]
    )


def kernel(text, emb_table, fc_w, fc_b):
    emb_lin = _linearize_table(emb_table)
    text_r = (
        (text & ~(TR_BLK - 1))
        + ((text & (TR_SUB - 1)) << 3)
        + ((text >> 10) & 7)
    )
    text2d = text_r.reshape(N_TOKENS // SUBCHUNK, SUBCHUNK)
    partials = _seg_sum_kernel(text2d, emb_lin)
    fc_w_p = jnp.concatenate([fc_w[:, 0::2], fc_w[:, 1::2]], axis=1)
    return pl.pallas_call(
        _head_body,
        out_shape=jax.ShapeDtypeStruct((BATCH, NUM_CLASS), jnp.float32),
    )(partials, fc_w_p, fc_b.reshape(1, NUM_CLASS))


# speedup vs baseline: 1.2422x; 1.0457x over previous
import functools

import jax
import jax.numpy as jnp
from jax import lax
from jax.experimental import pallas as pl
from jax.experimental.pallas import tpu as pltpu
from jax.experimental.pallas import tpu_sc as plsc

N_TOKENS = 1048576
VOCAB = 1000000
EMBED_DIM = 32
NUM_CLASS = 4
BATCH = 16
SEG = N_TOKENS // BATCH
HALF = SEG // 2
CHUNK = 2048
SUBCHUNK = 128
N_SUB = CHUNK // SUBCHUNK
N_STEPS = HALF // CHUNK
ROW_I32 = EMBED_DIM // 2

TR_BLK = 8192
TR_SUB = TR_BLK // 8
TR_I = 123


def _relayout_body(t_ref, o_ref):
    t = t_ref[...]
    tcat = jnp.concatenate(
        [t[:, m * TR_SUB:(m + 1) * TR_SUB] for m in range(8)], axis=0
    )
    rows = lax.broadcasted_iota(jnp.int32, (8 * EMBED_DIM, 4 * EMBED_DIM), 0)
    cols = lax.broadcasted_iota(jnp.int32, (8 * EMBED_DIM, 4 * EMBED_DIM), 1)
    s_even = (rows == 2 * cols).astype(jnp.bfloat16)
    s_odd = (rows == 2 * cols + 1).astype(jnp.bfloat16)
    tb = tcat.astype(jnp.bfloat16)
    dims = (((0,), (0,)), ((), ()))
    lo = lax.dot_general(tb, s_even, dims, preferred_element_type=jnp.float32)
    hi = lax.dot_general(tb, s_odd, dims, preferred_element_type=jnp.float32)

    def b16(f):
        return lax.shift_right_logical(
            lax.bitcast_convert_type(f, jnp.int32), 16
        )

    o_ref[...] = b16(lo) | lax.shift_left(b16(hi), 16)


def _linearize_table(emb_table):
    emb_t = emb_table.T
    lin = pl.pallas_call(
        _relayout_body,
        grid=(TR_I,),
        in_specs=[pl.BlockSpec((EMBED_DIM, TR_BLK), lambda i: (0, i))],
        out_specs=pl.BlockSpec((TR_SUB, 4 * EMBED_DIM), lambda i: (i, 0)),
        out_shape=jax.ShapeDtypeStruct((TR_I * TR_SUB, 4 * EMBED_DIM),
                                       jnp.int32),
    )(emb_t)
    return lin.reshape(TR_I * TR_BLK, ROW_I32)


def _seg_sum_kernel(text2d, emb_lin):
    mesh = plsc.VectorSubcoreMesh(core_axis_name="c", subcore_axis_name="s")

    @functools.partial(
        pl.kernel,
        mesh=mesh,
        out_type=jax.ShapeDtypeStruct((32, EMBED_DIM), jnp.float32),
        scratch_types=[
            pltpu.VMEM((2, N_SUB, SUBCHUNK), jnp.int32),
            pltpu.VMEM((2, CHUNK, ROW_I32), jnp.int32),
            pltpu.VMEM((EMBED_DIM,), jnp.float32),
            pltpu.SemaphoreType.DMA,
            pltpu.SemaphoreType.DMA,
        ],
        compiler_params=pltpu.CompilerParams(use_tc_tiling_on_sc=False),
    )
    def body(text_hbm, emb_hbm, out_hbm, idx_v, rows_v, acc_v, isem, gsem):
        c = lax.axis_index("c")
        s = lax.axis_index("s")
        wid = c * 16 + s
        row_base = s * (SEG // SUBCHUNK) + c * (HALF // SUBCHUNK)
        last = N_STEPS - 1

        def idx_copy(step, b):
            rb = row_base + lax.min(step, last) * N_SUB
            return pltpu.make_async_copy(
                text_hbm.at[pl.ds(rb, N_SUB)], idx_v.at[b], isem
            )

        def gather(b, j):
            return pltpu.make_async_copy(
                emb_hbm.at[idx_v.at[b].at[j]],
                rows_v.at[b].at[pl.ds(j * SUBCHUNK, SUBCHUNK)],
                gsem,
            )

        idx_copy(0, 0).start()
        idx_copy(0, 0).wait()
        for j in range(N_SUB):
            gather(0, j).start()
        idx_copy(1, 1).start()

        zero = jnp.zeros((16,), jnp.float32)

        def accum(b, carry):
            def rows8(r, c2):
                b0, b1 = c2
                base = r * 8
                for u in range(8):
                    w = rows_v[b, base + u, :]
                    ea = lax.bitcast_convert_type(
                        lax.shift_left(w, 16), jnp.float32
                    )
                    eb = lax.bitcast_convert_type(
                        w & jnp.int32(-65536), jnp.float32
                    )
                    b0 = b0 + ea
                    b1 = b1 + eb
                return b0, b1

            return lax.fori_loop(0, CHUNK // 8, rows8, carry)

        def pair(g, carry):
            for b in (0, 1):
                i = g * 2 + b
                idx_copy(i + 1, 1 - b).wait()
                for j in range(N_SUB):
                    gather(1 - b, j).start()
                for j in range(N_SUB):
                    gather(b, j).wait()
                idx_copy(i + 2, b).start()
                carry = accum(b, carry)
            return carry

        a0, a1 = lax.fori_loop(0, N_STEPS // 2, pair, (zero, zero))

        idx_copy(N_STEPS + 1, 1).wait()
        for j in range(N_SUB):
            gather(0, j).wait()

        acc_v[pl.ds(0, 16)] = a0
        acc_v[pl.ds(16, 16)] = a1
        pltpu.sync_copy(acc_v, out_hbm.at[wid])

    return body(text2d, emb_lin)


def _head_body(p_ref, w_ref, b_ref, o_ref):
    p = p_ref[...]
    pooled = (p[0:16, :] + p[16:32, :]) * (1.0 / SEG)
    o_ref[...] = (
        lax.dot_general(
            pooled, w_ref[...], (((1,), (1,)), ((), ())),
            preferred_element_type=jnp.float32,
        )
        + b_ref[...]
    )


def kernel(text, emb_table, fc_w, fc_b):
    emb_lin = _linearize_table(emb_table)
    text_r = (
        (text & ~(TR_BLK - 1))
        + ((text & (TR_SUB - 1)) << 3)
        + ((text >> 10) & 7)
    )
    text2d = text_r.reshape(N_TOKENS // SUBCHUNK, SUBCHUNK)
    partials = _seg_sum_kernel(text2d, emb_lin)
    fc_w_p = jnp.concatenate([fc_w[:, 0::2], fc_w[:, 1::2]], axis=1)
    return pl.pallas_call(
        _head_body,
        out_shape=jax.ShapeDtypeStruct((BATCH, NUM_CLASS), jnp.float32),
    )(partials, fc_w_p, fc_b.reshape(1, NUM_CLASS))


# speedup vs baseline: 1.4930x; 1.2018x over previous
import functools

import jax
import jax.numpy as jnp
from jax import lax
from jax.experimental import pallas as pl
from jax.experimental.pallas import tpu as pltpu
from jax.experimental.pallas import tpu_sc as plsc

N_TOKENS = 1048576
VOCAB = 1000000
EMBED_DIM = 32
NUM_CLASS = 4
BATCH = 16
SEG = N_TOKENS // BATCH
HALF = SEG // 2
CHUNK = 2048
SUBCHUNK = 128
N_SUB = CHUNK // SUBCHUNK
N_STEPS = HALF // CHUNK
ROW_I32 = EMBED_DIM // 2

TR_BLK = 16384
TR_SUB = TR_BLK // 8
TR_I = -(-VOCAB // TR_BLK)
TR_SHIFT = TR_SUB.bit_length() - 1


def _relayout_body(t_ref, o_ref):
    t = t_ref[...]
    tcat = jnp.concatenate(
        [t[:, m * TR_SUB:(m + 1) * TR_SUB] for m in range(8)], axis=0
    )
    rows = lax.broadcasted_iota(jnp.int32, (8 * EMBED_DIM, 4 * EMBED_DIM), 0)
    cols = lax.broadcasted_iota(jnp.int32, (8 * EMBED_DIM, 4 * EMBED_DIM), 1)
    s_even = (rows == 2 * cols).astype(jnp.bfloat16)
    s_odd = (rows == 2 * cols + 1).astype(jnp.bfloat16)
    tb = tcat.astype(jnp.bfloat16)
    dims = (((0,), (0,)), ((), ()))
    lo = lax.dot_general(tb, s_even, dims, preferred_element_type=jnp.float32)
    hi = lax.dot_general(tb, s_odd, dims, preferred_element_type=jnp.float32)

    def b16(f):
        return lax.shift_right_logical(
            lax.bitcast_convert_type(f, jnp.int32), 16
        )

    o_ref[...] = b16(lo) | lax.shift_left(b16(hi), 16)


def _linearize_table(emb_table):
    emb_t = emb_table.T
    lin = pl.pallas_call(
        _relayout_body,
        grid=(TR_I,),
        in_specs=[pl.BlockSpec((EMBED_DIM, TR_BLK), lambda i: (0, i))],
        out_specs=pl.BlockSpec((TR_SUB, 4 * EMBED_DIM), lambda i: (i, 0)),
        out_shape=jax.ShapeDtypeStruct((TR_I * TR_SUB, 4 * EMBED_DIM),
                                       jnp.int32),
    )(emb_t)
    return lin.reshape(TR_I * TR_BLK, ROW_I32)


def _seg_sum_kernel(text2d, emb_lin):
    mesh = plsc.VectorSubcoreMesh(core_axis_name="c", subcore_axis_name="s")

    @functools.partial(
        pl.kernel,
        mesh=mesh,
        out_type=jax.ShapeDtypeStruct((32, EMBED_DIM), jnp.float32),
        scratch_types=[
            pltpu.VMEM((2, N_SUB, SUBCHUNK), jnp.int32),
            pltpu.VMEM((2, CHUNK, ROW_I32), jnp.int32),
            pltpu.VMEM((EMBED_DIM,), jnp.float32),
            pltpu.SemaphoreType.DMA,
            pltpu.SemaphoreType.DMA,
        ],
        compiler_params=pltpu.CompilerParams(use_tc_tiling_on_sc=False),
    )
    def body(text_hbm, emb_hbm, out_hbm, idx_v, rows_v, acc_v, isem, gsem):
        c = lax.axis_index("c")
        s = lax.axis_index("s")
        wid = c * 16 + s
        row_base = s * (SEG // SUBCHUNK) + c * (HALF // SUBCHUNK)
        last = N_STEPS - 1

        def idx_copy(step, b):
            rb = row_base + lax.min(step, last) * N_SUB
            return pltpu.make_async_copy(
                text_hbm.at[pl.ds(rb, N_SUB)], idx_v.at[b], isem
            )

        def gather(b, j):
            return pltpu.make_async_copy(
                emb_hbm.at[idx_v.at[b].at[j]],
                rows_v.at[b].at[pl.ds(j * SUBCHUNK, SUBCHUNK)],
                gsem,
            )

        idx_copy(0, 0).start()
        idx_copy(0, 0).wait()
        for j in range(N_SUB):
            gather(0, j).start()
        idx_copy(1, 1).start()

        zero = jnp.zeros((16,), jnp.float32)

        def accum(b, carry):
            def rows8(r, c2):
                b0, b1 = c2
                base = r * 8
                for u in range(8):
                    w = rows_v[b, base + u, :]
                    ea = lax.bitcast_convert_type(
                        lax.shift_left(w, 16), jnp.float32
                    )
                    eb = lax.bitcast_convert_type(
                        w & jnp.int32(-65536), jnp.float32
                    )
                    b0 = b0 + ea
                    b1 = b1 + eb
                return b0, b1

            return lax.fori_loop(0, CHUNK // 8, rows8, carry)

        def pair(g, carry):
            for b in (0, 1):
                i = g * 2 + b
                idx_copy(i + 1, 1 - b).wait()
                for j in range(N_SUB):
                    gather(1 - b, j).start()
                for j in range(N_SUB):
                    gather(b, j).wait()
                idx_copy(i + 2, b).start()
                carry = accum(b, carry)
            return carry

        a0, a1 = lax.fori_loop(0, N_STEPS // 2, pair, (zero, zero))

        idx_copy(N_STEPS + 1, 1).wait()
        for j in range(N_SUB):
            gather(0, j).wait()

        acc_v[pl.ds(0, 16)] = a0
        acc_v[pl.ds(16, 16)] = a1
        pltpu.sync_copy(acc_v, out_hbm.at[wid])

    return body(text2d, emb_lin)


def _head_body(p_ref, w_ref, b_ref, o_ref):
    p = p_ref[...]
    pooled = (p[0:16, :] + p[16:32, :]) * (1.0 / SEG)
    o_ref[...] = (
        lax.dot_general(
            pooled, w_ref[...], (((1,), (1,)), ((), ())),
            preferred_element_type=jnp.float32,
        )
        + b_ref[...]
    )


def kernel(text, emb_table, fc_w, fc_b):
    emb_lin = _linearize_table(emb_table)
    text_r = (
        (text & ~(TR_BLK - 1))
        + ((text & (TR_SUB - 1)) << 3)
        + ((text >> TR_SHIFT) & 7)
    )
    text2d = text_r.reshape(N_TOKENS // SUBCHUNK, SUBCHUNK)
    partials = _seg_sum_kernel(text2d, emb_lin)
    fc_w_p = jnp.concatenate([fc_w[:, 0::2], fc_w[:, 1::2]], axis=1)
    return pl.pallas_call(
        _head_body,
        out_shape=jax.ShapeDtypeStruct((BATCH, NUM_CLASS), jnp.float32),
    )(partials, fc_w_p, fc_b.reshape(1, NUM_CLASS))


# speedup vs baseline: 1.6982x; 1.1375x over previous
import functools

import jax
import jax.numpy as jnp
from jax import lax
from jax.experimental import pallas as pl
from jax.experimental.pallas import tpu as pltpu
from jax.experimental.pallas import tpu_sc as plsc

N_TOKENS = 1048576
VOCAB = 1000000
EMBED_DIM = 32
NUM_CLASS = 4
BATCH = 16
SEG = N_TOKENS // BATCH
HALF = SEG // 2
CHUNK = 2048
SUBCHUNK = 128
N_SUB = CHUNK // SUBCHUNK
N_STEPS = HALF // CHUNK
ROW_I32 = EMBED_DIM // 2

TR_BLK = 32768
TR_SUB = TR_BLK // 8
TR_I = -(-VOCAB // TR_BLK)
TR_SHIFT = TR_SUB.bit_length() - 1


def _relayout_body(t_ref, o_ref):
    t = t_ref[...]
    tcat = jnp.concatenate(
        [t[:, m * TR_SUB:(m + 1) * TR_SUB] for m in range(8)], axis=0
    )
    rows = lax.broadcasted_iota(jnp.int32, (8 * EMBED_DIM, 4 * EMBED_DIM), 0)
    cols = lax.broadcasted_iota(jnp.int32, (8 * EMBED_DIM, 4 * EMBED_DIM), 1)
    s_even = (rows == 2 * cols).astype(jnp.bfloat16)
    s_odd = (rows == 2 * cols + 1).astype(jnp.bfloat16)
    tb = tcat.astype(jnp.bfloat16)
    dims = (((0,), (0,)), ((), ()))
    lo = lax.dot_general(tb, s_even, dims, preferred_element_type=jnp.float32)
    hi = lax.dot_general(tb, s_odd, dims, preferred_element_type=jnp.float32)

    def b16(f):
        return lax.shift_right_logical(
            lax.bitcast_convert_type(f, jnp.int32), 16
        )

    o_ref[...] = b16(lo) | lax.shift_left(b16(hi), 16)


def _linearize_table(emb_table):
    emb_t = emb_table.T
    lin = pl.pallas_call(
        _relayout_body,
        grid=(TR_I,),
        in_specs=[pl.BlockSpec((EMBED_DIM, TR_BLK), lambda i: (0, i))],
        out_specs=pl.BlockSpec((TR_SUB, 4 * EMBED_DIM), lambda i: (i, 0)),
        out_shape=jax.ShapeDtypeStruct((TR_I * TR_SUB, 4 * EMBED_DIM),
                                       jnp.int32),
    )(emb_t)
    return lin.reshape(TR_I * TR_BLK, ROW_I32)


def _seg_sum_kernel(text2d, emb_lin):
    mesh = plsc.VectorSubcoreMesh(core_axis_name="c", subcore_axis_name="s")

    @functools.partial(
        pl.kernel,
        mesh=mesh,
        out_type=jax.ShapeDtypeStruct((32, EMBED_DIM), jnp.float32),
        scratch_types=[
            pltpu.VMEM((2, N_SUB, SUBCHUNK), jnp.int32),
            pltpu.VMEM((2, CHUNK, ROW_I32), jnp.int32),
            pltpu.VMEM((EMBED_DIM,), jnp.float32),
            pltpu.SemaphoreType.DMA,
            pltpu.SemaphoreType.DMA,
        ],
        compiler_params=pltpu.CompilerParams(use_tc_tiling_on_sc=False),
    )
    def body(text_hbm, emb_hbm, out_hbm, idx_v, rows_v, acc_v, isem, gsem):
        c = lax.axis_index("c")
        s = lax.axis_index("s")
        wid = c * 16 + s
        row_base = s * (SEG // SUBCHUNK) + c * (HALF // SUBCHUNK)
        last = N_STEPS - 1

        def idx_copy(step, b):
            rb = row_base + lax.min(step, last) * N_SUB
            return pltpu.make_async_copy(
                text_hbm.at[pl.ds(rb, N_SUB)], idx_v.at[b], isem
            )

        def gather(b, j):
            return pltpu.make_async_copy(
                emb_hbm.at[idx_v.at[b].at[j]],
                rows_v.at[b].at[pl.ds(j * SUBCHUNK, SUBCHUNK)],
                gsem,
            )

        idx_copy(0, 0).start()
        idx_copy(0, 0).wait()
        for j in range(N_SUB):
            gather(0, j).start()
        idx_copy(1, 1).start()

        zero = jnp.zeros((16,), jnp.float32)

        def accum(b, carry):
            def rows8(r, c2):
                b0, b1 = c2
                base = r * 8
                for u in range(8):
                    w = rows_v[b, base + u, :]
                    ea = lax.bitcast_convert_type(
                        lax.shift_left(w, 16), jnp.float32
                    )
                    eb = lax.bitcast_convert_type(
                        w & jnp.int32(-65536), jnp.float32
                    )
                    b0 = b0 + ea
                    b1 = b1 + eb
                return b0, b1

            return lax.fori_loop(0, CHUNK // 8, rows8, carry)

        def pair(g, carry):
            for b in (0, 1):
                i = g * 2 + b
                idx_copy(i + 1, 1 - b).wait()
                for j in range(N_SUB):
                    gather(1 - b, j).start()
                for j in range(N_SUB):
                    gather(b, j).wait()
                idx_copy(i + 2, b).start()
                carry = accum(b, carry)
            return carry

        a0, a1 = lax.fori_loop(0, N_STEPS // 2, pair, (zero, zero))

        idx_copy(N_STEPS + 1, 1).wait()
        for j in range(N_SUB):
            gather(0, j).wait()

        acc_v[pl.ds(0, 16)] = a0
        acc_v[pl.ds(16, 16)] = a1
        pltpu.sync_copy(acc_v, out_hbm.at[wid])

    return body(text2d, emb_lin)


def _head_body(p_ref, w_ref, b_ref, o_ref):
    p = p_ref[...]
    pooled = (p[0:16, :] + p[16:32, :]) * (1.0 / SEG)
    o_ref[...] = (
        lax.dot_general(
            pooled, w_ref[...], (((1,), (1,)), ((), ())),
            preferred_element_type=jnp.float32,
        )
        + b_ref[...]
    )


def kernel(text, emb_table, fc_w, fc_b):
    emb_lin = _linearize_table(emb_table)
    text_r = (
        (text & ~(TR_BLK - 1))
        + ((text & (TR_SUB - 1)) << 3)
        + ((text >> TR_SHIFT) & 7)
    )
    text2d = text_r.reshape(N_TOKENS // SUBCHUNK, SUBCHUNK)
    partials = _seg_sum_kernel(text2d, emb_lin)
    fc_w_p = jnp.concatenate([fc_w[:, 0::2], fc_w[:, 1::2]], axis=1)
    return pl.pallas_call(
        _head_body,
        out_shape=jax.ShapeDtypeStruct((BATCH, NUM_CLASS), jnp.float32),
    )(partials, fc_w_p, fc_b.reshape(1, NUM_CLASS))


# speedup vs baseline: 1.7265x; 1.0167x over previous
import functools

import jax
import jax.numpy as jnp
from jax import lax
from jax.experimental import pallas as pl
from jax.experimental.pallas import tpu as pltpu
from jax.experimental.pallas import tpu_sc as plsc

N_TOKENS = 1048576
VOCAB = 1000000
EMBED_DIM = 32
NUM_CLASS = 4
BATCH = 16
SEG = N_TOKENS // BATCH
HALF = SEG // 2
CHUNK = 2048
SUBCHUNK = 128
N_SUB = CHUNK // SUBCHUNK
N_STEPS = HALF // CHUNK
ROW_I32 = EMBED_DIM // 2

TR_BLK = 65536
TR_SUB = TR_BLK // 8
TR_I = -(-VOCAB // TR_BLK)
TR_SHIFT = TR_SUB.bit_length() - 1


def _relayout_body(t_ref, o_ref):
    t = t_ref[...]
    tcat = jnp.concatenate(
        [t[:, m * TR_SUB:(m + 1) * TR_SUB] for m in range(8)], axis=0
    )
    rows = lax.broadcasted_iota(jnp.int32, (8 * EMBED_DIM, 4 * EMBED_DIM), 0)
    cols = lax.broadcasted_iota(jnp.int32, (8 * EMBED_DIM, 4 * EMBED_DIM), 1)
    s_even = (rows == 2 * cols).astype(jnp.bfloat16)
    s_odd = (rows == 2 * cols + 1).astype(jnp.bfloat16)
    tb = tcat.astype(jnp.bfloat16)
    dims = (((0,), (0,)), ((), ()))
    lo = lax.dot_general(tb, s_even, dims, preferred_element_type=jnp.float32)
    hi = lax.dot_general(tb, s_odd, dims, preferred_element_type=jnp.float32)

    def b16(f):
        return lax.shift_right_logical(
            lax.bitcast_convert_type(f, jnp.int32), 16
        )

    o_ref[...] = b16(lo) | lax.shift_left(b16(hi), 16)


def _linearize_table(emb_table):
    emb_t = emb_table.T
    lin = pl.pallas_call(
        _relayout_body,
        grid=(TR_I,),
        in_specs=[pl.BlockSpec((EMBED_DIM, TR_BLK), lambda i: (0, i))],
        out_specs=pl.BlockSpec((TR_SUB, 4 * EMBED_DIM), lambda i: (i, 0)),
        out_shape=jax.ShapeDtypeStruct((TR_I * TR_SUB, 4 * EMBED_DIM),
                                       jnp.int32),
    )(emb_t)
    return lin.reshape(TR_I * TR_BLK, ROW_I32)


def _seg_sum_kernel(text2d, emb_lin):
    mesh = plsc.VectorSubcoreMesh(core_axis_name="c", subcore_axis_name="s")

    @functools.partial(
        pl.kernel,
        mesh=mesh,
        out_type=jax.ShapeDtypeStruct((32, EMBED_DIM), jnp.float32),
        scratch_types=[
            pltpu.VMEM((2, N_SUB, SUBCHUNK), jnp.int32),
            pltpu.VMEM((2, CHUNK, ROW_I32), jnp.int32),
            pltpu.VMEM((EMBED_DIM,), jnp.float32),
            pltpu.SemaphoreType.DMA,
            pltpu.SemaphoreType.DMA,
        ],
        compiler_params=pltpu.CompilerParams(use_tc_tiling_on_sc=False),
    )
    def body(text_hbm, emb_hbm, out_hbm, idx_v, rows_v, acc_v, isem, gsem):
        c = lax.axis_index("c")
        s = lax.axis_index("s")
        wid = c * 16 + s
        row_base = s * (SEG // SUBCHUNK) + c * (HALF // SUBCHUNK)
        last = N_STEPS - 1

        def idx_copy(step, b):
            rb = row_base + lax.min(step, last) * N_SUB
            return pltpu.make_async_copy(
                text_hbm.at[pl.ds(rb, N_SUB)], idx_v.at[b], isem
            )

        def gather(b, j):
            return pltpu.make_async_copy(
                emb_hbm.at[idx_v.at[b].at[j]],
                rows_v.at[b].at[pl.ds(j * SUBCHUNK, SUBCHUNK)],
                gsem,
            )

        idx_copy(0, 0).start()
        idx_copy(0, 0).wait()
        for j in range(N_SUB):
            gather(0, j).start()
        idx_copy(1, 1).start()

        zero = jnp.zeros((16,), jnp.float32)

        def accum(b, carry):
            def rows8(r, c2):
                b0, b1 = c2
                base = r * 8
                for u in range(8):
                    w = rows_v[b, base + u, :]
                    ea = lax.bitcast_convert_type(
                        lax.shift_left(w, 16), jnp.float32
                    )
                    eb = lax.bitcast_convert_type(
                        w & jnp.int32(-65536), jnp.float32
                    )
                    b0 = b0 + ea
                    b1 = b1 + eb
                return b0, b1

            return lax.fori_loop(0, CHUNK // 8, rows8, carry)

        def pair(g, carry):
            for b in (0, 1):
                i = g * 2 + b
                idx_copy(i + 1, 1 - b).wait()
                for j in range(N_SUB):
                    gather(1 - b, j).start()
                for j in range(N_SUB):
                    gather(b, j).wait()
                idx_copy(i + 2, b).start()
                carry = accum(b, carry)
            return carry

        a0, a1 = lax.fori_loop(0, N_STEPS // 2, pair, (zero, zero))

        idx_copy(N_STEPS + 1, 1).wait()
        for j in range(N_SUB):
            gather(0, j).wait()

        acc_v[pl.ds(0, 16)] = a0
        acc_v[pl.ds(16, 16)] = a1
        pltpu.sync_copy(acc_v, out_hbm.at[wid])

    return body(text2d, emb_lin)


def _head_body(p_ref, w_ref, b_ref, o_ref):
    p = p_ref[...]
    pooled = (p[0:16, :] + p[16:32, :]) * (1.0 / SEG)
    o_ref[...] = (
        lax.dot_general(
            pooled, w_ref[...], (((1,), (1,)), ((), ())),
            preferred_element_type=jnp.float32,
        )
        + b_ref[...]
    )


def kernel(text, emb_table, fc_w, fc_b):
    emb_lin = _linearize_table(emb_table)
    text_r = (
        (text & ~(TR_BLK - 1))
        + ((text & (TR_SUB - 1)) << 3)
        + ((text >> TR_SHIFT) & 7)
    )
    text2d = text_r.reshape(N_TOKENS // SUBCHUNK, SUBCHUNK)
    partials = _seg_sum_kernel(text2d, emb_lin)
    fc_w_p = jnp.concatenate([fc_w[:, 0::2], fc_w[:, 1::2]], axis=1)
    return pl.pallas_call(
        _head_body,
        out_shape=jax.ShapeDtypeStruct((BATCH, NUM_CLASS), jnp.float32),
    )(partials, fc_w_p, fc_b.reshape(1, NUM_CLASS))


# speedup vs baseline: 1.7692x; 1.0248x over previous
import functools

import jax
import jax.numpy as jnp
from jax import lax
from jax.experimental import pallas as pl
from jax.experimental.pallas import tpu as pltpu
from jax.experimental.pallas import tpu_sc as plsc

N_TOKENS = 1048576
VOCAB = 1000000
EMBED_DIM = 32
NUM_CLASS = 4
BATCH = 16
SEG = N_TOKENS // BATCH
HALF = SEG // 2
CHUNK = 2048
SUBCHUNK = 128
N_SUB = CHUNK // SUBCHUNK
N_STEPS = HALF // CHUNK
ROW_I32 = EMBED_DIM // 2

TR_BLK = 131072
TR_SUB = TR_BLK // 8
TR_I = -(-VOCAB // TR_BLK)
TR_SHIFT = TR_SUB.bit_length() - 1


def _relayout_body(t_ref, o_ref):
    t = t_ref[...]
    tcat = jnp.concatenate(
        [t[:, m * TR_SUB:(m + 1) * TR_SUB] for m in range(8)], axis=0
    )
    rows = lax.broadcasted_iota(jnp.int32, (8 * EMBED_DIM, 4 * EMBED_DIM), 0)
    cols = lax.broadcasted_iota(jnp.int32, (8 * EMBED_DIM, 4 * EMBED_DIM), 1)
    s_even = (rows == 2 * cols).astype(jnp.bfloat16)
    s_odd = (rows == 2 * cols + 1).astype(jnp.bfloat16)
    tb = tcat.astype(jnp.bfloat16)
    dims = (((0,), (0,)), ((), ()))
    lo = lax.dot_general(tb, s_even, dims, preferred_element_type=jnp.float32)
    hi = lax.dot_general(tb, s_odd, dims, preferred_element_type=jnp.float32)

    def b16(f):
        return lax.shift_right_logical(
            lax.bitcast_convert_type(f, jnp.int32), 16
        )

    o_ref[...] = b16(lo) | lax.shift_left(b16(hi), 16)


def _linearize_table(emb_table):
    emb_t = emb_table.T
    lin = pl.pallas_call(
        _relayout_body,
        grid=(TR_I,),
        in_specs=[pl.BlockSpec((EMBED_DIM, TR_BLK), lambda i: (0, i))],
        out_specs=pl.BlockSpec((TR_SUB, 4 * EMBED_DIM), lambda i: (i, 0)),
        out_shape=jax.ShapeDtypeStruct((TR_I * TR_SUB, 4 * EMBED_DIM),
                                       jnp.int32),
    )(emb_t)
    return lin.reshape(TR_I * TR_BLK, ROW_I32)


def _seg_sum_kernel(text2d, emb_lin):
    mesh = plsc.VectorSubcoreMesh(core_axis_name="c", subcore_axis_name="s")

    @functools.partial(
        pl.kernel,
        mesh=mesh,
        out_type=jax.ShapeDtypeStruct((32, EMBED_DIM), jnp.float32),
        scratch_types=[
            pltpu.VMEM((2, N_SUB, SUBCHUNK), jnp.int32),
            pltpu.VMEM((2, CHUNK, ROW_I32), jnp.int32),
            pltpu.VMEM((EMBED_DIM,), jnp.float32),
            pltpu.SemaphoreType.DMA,
            pltpu.SemaphoreType.DMA,
        ],
        compiler_params=pltpu.CompilerParams(use_tc_tiling_on_sc=False),
    )
    def body(text_hbm, emb_hbm, out_hbm, idx_v, rows_v, acc_v, isem, gsem):
        c = lax.axis_index("c")
        s = lax.axis_index("s")
        wid = c * 16 + s
        row_base = s * (SEG // SUBCHUNK) + c * (HALF // SUBCHUNK)
        last = N_STEPS - 1

        def idx_copy(step, b):
            rb = row_base + lax.min(step, last) * N_SUB
            return pltpu.make_async_copy(
                text_hbm.at[pl.ds(rb, N_SUB)], idx_v.at[b], isem
            )

        def gather(b, j):
            return pltpu.make_async_copy(
                emb_hbm.at[idx_v.at[b].at[j]],
                rows_v.at[b].at[pl.ds(j * SUBCHUNK, SUBCHUNK)],
                gsem,
            )

        idx_copy(0, 0).start()
        idx_copy(0, 0).wait()
        for j in range(N_SUB):
            gather(0, j).start()
        idx_copy(1, 1).start()

        zero = jnp.zeros((16,), jnp.float32)

        def accum(b, carry):
            def rows8(r, c2):
                b0, b1 = c2
                base = r * 8
                for u in range(8):
                    w = rows_v[b, base + u, :]
                    ea = lax.bitcast_convert_type(
                        lax.shift_left(w, 16), jnp.float32
                    )
                    eb = lax.bitcast_convert_type(
                        w & jnp.int32(-65536), jnp.float32
                    )
                    b0 = b0 + ea
                    b1 = b1 + eb
                return b0, b1

            return lax.fori_loop(0, CHUNK // 8, rows8, carry)

        def pair(g, carry):
            for b in (0, 1):
                i = g * 2 + b
                idx_copy(i + 1, 1 - b).wait()
                for j in range(N_SUB):
                    gather(1 - b, j).start()
                for j in range(N_SUB):
                    gather(b, j).wait()
                idx_copy(i + 2, b).start()
                carry = accum(b, carry)
            return carry

        a0, a1 = lax.fori_loop(0, N_STEPS // 2, pair, (zero, zero))

        idx_copy(N_STEPS + 1, 1).wait()
        for j in range(N_SUB):
            gather(0, j).wait()

        acc_v[pl.ds(0, 16)] = a0
        acc_v[pl.ds(16, 16)] = a1
        pltpu.sync_copy(acc_v, out_hbm.at[wid])

    return body(text2d, emb_lin)


def _head_body(p_ref, w_ref, b_ref, o_ref):
    p = p_ref[...]
    pooled = (p[0:16, :] + p[16:32, :]) * (1.0 / SEG)
    o_ref[...] = (
        lax.dot_general(
            w_ref[...], pooled, (((1,), (1,)), ((), ())),
            preferred_element_type=jnp.float32,
        )
        + b_ref[...]
    )


def kernel(text, emb_table, fc_w, fc_b):
    emb_lin = _linearize_table(emb_table)
    text_r = (
        (text & ~(TR_BLK - 1))
        + ((text & (TR_SUB - 1)) << 3)
        + ((text >> TR_SHIFT) & 7)
    )
    text2d = text_r.reshape(N_TOKENS // SUBCHUNK, SUBCHUNK)
    partials = _seg_sum_kernel(text2d, emb_lin)
    fc_w_p = jnp.concatenate([fc_w[:, 0::2], fc_w[:, 1::2]], axis=1)
    out_t = pl.pallas_call(
        _head_body,
        out_shape=jax.ShapeDtypeStruct((NUM_CLASS, BATCH), jnp.float32),
    )(partials, fc_w_p, fc_b.reshape(NUM_CLASS, 1))
    return out_t.T
